# Initial kernel scaffold; baseline (speedup 1.0000x reference)
import jax
import jax.numpy as jnp
from jax.experimental import pallas as pl


def kernel(locations, box_cls, box_regression, centerness, image_sizes):
    raise NotImplementedError("write your pallas kernel here")


# speedup vs baseline: 1.0131x; 1.0131x over previous
import jax
import jax.numpy as jnp
from jax.experimental import pallas as pl

PRE_NMS_THRESH = 0.01
PRE_NMS_TOP_N = 1000
FPN_POST_NMS_TOP_N = 100


def _score_body(cls_ref, ctr_ref, out_ref):
    c = cls_ref[...]
    s = jax.nn.sigmoid(c)
    ct = jax.nn.sigmoid(ctr_ref[...])
    out_ref[...] = jnp.where(s > PRE_NMS_THRESH, s * ct, -1.0)


def kernel(locations, box_cls, box_regression, centerness, image_sizes):
    N, C, H, W = box_cls.shape
    HW = H * W
    clsT = jnp.transpose(box_cls, (0, 2, 3, 1)).reshape(N, HW, C)
    ctrT = jnp.transpose(centerness, (0, 2, 3, 1)).reshape(N, HW, 1)
    reg = jnp.transpose(box_regression, (0, 2, 3, 1)).reshape(N, HW, 4)

    blk = 2592
    masked = pl.pallas_call(
        _score_body,
        grid=(N, HW // blk),
        in_specs=[
            pl.BlockSpec((1, blk, C), lambda n, i: (n, i, 0)),
            pl.BlockSpec((1, blk, 1), lambda n, i: (n, i, 0)),
        ],
        out_specs=pl.BlockSpec((1, blk, C), lambda n, i: (n, i, 0)),
        out_shape=jax.ShapeDtypeStruct((N, HW, C), jnp.float32),
    )(clsT, ctrT)

    flat = masked.reshape(N, HW * C)
    top_vals, top_idx = jax.lax.top_k(flat, PRE_NMS_TOP_N)
    loc_idx = top_idx // C
    labels = (top_idx % C) + 1
    per_reg = jnp.take_along_axis(reg, loc_idx[:, :, None], axis=1)
    per_loc = locations[loc_idx]
    x1 = per_loc[..., 0] - per_reg[..., 0]
    y1 = per_loc[..., 1] - per_reg[..., 1]
    x2 = per_loc[..., 0] + per_reg[..., 2]
    y2 = per_loc[..., 1] + per_reg[..., 3]
    w = jnp.maximum(image_sizes[:, 1], 2).astype(jnp.float32)[:, None]
    h = jnp.maximum(image_sizes[:, 0], 2).astype(jnp.float32)[:, None]
    x1 = jnp.clip(x1, 0.0, w - 1.0)
    x2 = jnp.clip(x2, 0.0, w - 1.0)
    y1 = jnp.clip(y1, 0.0, h - 1.0)
    y2 = jnp.clip(y2, 0.0, h - 1.0)
    ws = x2 - x1 + 1.0
    hs = y2 - y1 + 1.0
    keep = (ws >= 0) & (hs >= 0) & (top_vals > 0.0)
    final_scores = jnp.where(keep, top_vals, -1.0)
    fin_vals, fin_idx = jax.lax.top_k(final_scores, FPN_POST_NMS_TOP_N)
    boxes = jnp.stack([x1, y1, x2, y2], axis=-1)
    fin_boxes = jnp.take_along_axis(boxes, fin_idx[:, :, None], axis=1)
    fin_labels = jnp.take_along_axis(labels, fin_idx, axis=1).astype(jnp.float32)
    out = jnp.concatenate([fin_boxes, fin_vals[:, :, None], fin_labels[:, :, None]], axis=-1)
    return out
